# baseline (speedup 1.0000x reference)
import functools
import math

import jax
import jax.numpy as jnp
from jax import lax
from jax.experimental import pallas as pl
from jax.experimental.pallas import tpu as pltpu
from jax.experimental.pallas import tpu_sc as plsc

D_MODEL = 1024
SEQ = 4096
LANES = 16
NUM_CORES = 2
NUM_SUBCORES = 16
NW = NUM_CORES * NUM_SUBCORES
B_PER_W = SEQ // NW
EDGE = 16
TAIL = 8
MID = 32
N_MID = 3
COLS = D_MODEL // LANES
CBLK = 8
SCALE = math.sqrt(float(D_MODEL))


def _make_emb_kernel():
    mesh = plsc.VectorSubcoreMesh(core_axis_name="c", subcore_axis_name="s")

    @functools.partial(
        pl.kernel,
        mesh=mesh,
        out_type=jax.ShapeDtypeStruct((SEQ, 1, D_MODEL), jnp.float32),
        scratch_types=[
            pltpu.VMEM((B_PER_W,), jnp.int32),
            pltpu.VMEM((1, D_MODEL), jnp.float32),
            pltpu.VMEM((EDGE, D_MODEL), jnp.float32),
            pltpu.VMEM((TAIL, D_MODEL), jnp.float32),
            pltpu.VMEM((N_MID * MID, D_MODEL), jnp.float32),
            pltpu.SemaphoreType.DMA,
            pltpu.SemaphoreType.DMA,
            pltpu.SemaphoreType.DMA,
            pltpu.SemaphoreType.DMA,
            pltpu.SemaphoreType.DMA((N_MID,)),
            pltpu.SemaphoreType.DMA((N_MID,)),
        ],
    )
    def emb_kernel(x_hbm, w_hbm, pe_hbm, out_hbm, idx_v, pe_v, edge, tbuf,
                   arena, esem, oesem, tsem, otsem, gsems, osems):
        wid = lax.axis_index("s") * NUM_CORES + lax.axis_index("c")
        base = wid * B_PER_W
        pltpu.sync_copy(x_hbm.at[pl.ds(base, B_PER_W)], idx_v)

        def compute(buf, row0, nrows):
            def blk(ib, carry):
                col0 = ib * (CBLK * LANES)
                pv = [
                    pe_v[0, pl.ds(col0 + i * LANES, LANES)]
                    for i in range(CBLK)
                ]

                def body(j, carry2):
                    for i in range(CBLK):
                        sl = pl.ds(col0 + i * LANES, LANES)
                        buf[row0 + j, sl] = buf[row0 + j, sl] * SCALE + pv[i]
                    return carry2

                lax.fori_loop(0, nrows, body, 0)
                return carry

            lax.fori_loop(0, COLS // CBLK, blk, 0)

        g_head = pltpu.async_copy(
            w_hbm.at[idx_v.at[pl.ds(0, EDGE)]], edge, esem
        )
        for k in range(N_MID):
            pltpu.async_copy(
                w_hbm.at[idx_v.at[pl.ds(EDGE + k * MID, MID)]],
                arena.at[pl.ds(k * MID, MID)],
                gsems.at[k],
            )
        g_t1 = pltpu.async_copy(
            w_hbm.at[idx_v.at[pl.ds(B_PER_W - 2 * TAIL, TAIL)]], tbuf, tsem
        )
        pltpu.sync_copy(pe_hbm.at[pl.ds(0, 1), 0], pe_v)

        g_head.wait()
        compute(edge, 0, EDGE)
        pltpu.async_copy(edge, out_hbm.at[pl.ds(base, EDGE), 0], oesem)

        def mid(k, carry):
            row0 = k * MID
            pltpu.make_async_copy(
                w_hbm.at[idx_v.at[pl.ds(0, MID)]],
                arena.at[pl.ds(row0, MID)],
                gsems.at[k],
            ).wait()
            compute(arena, row0, MID)
            pltpu.async_copy(
                arena.at[pl.ds(row0, MID)],
                out_hbm.at[pl.ds(base + EDGE + row0, MID), 0],
                osems.at[k],
            )

            @pl.when(k == 0)
            def _():
                pltpu.make_async_copy(
                    edge, out_hbm.at[pl.ds(base, EDGE), 0], oesem
                ).wait()
                pltpu.async_copy(
                    w_hbm.at[idx_v.at[pl.ds(B_PER_W - TAIL, TAIL)]],
                    edge.at[pl.ds(0, TAIL)],
                    esem,
                )

            return carry

        lax.fori_loop(0, N_MID, mid, 0)

        g_t1.wait()
        compute(tbuf, 0, TAIL)
        pltpu.async_copy(
            tbuf, out_hbm.at[pl.ds(base + B_PER_W - 2 * TAIL, TAIL), 0], otsem
        )
        pltpu.make_async_copy(
            w_hbm.at[idx_v.at[pl.ds(B_PER_W - TAIL, TAIL)]],
            edge.at[pl.ds(0, TAIL)],
            esem,
        ).wait()
        compute(edge, 0, TAIL)
        pltpu.async_copy(
            edge.at[pl.ds(0, TAIL)],
            out_hbm.at[pl.ds(base + B_PER_W - TAIL, TAIL), 0],
            oesem,
        ).wait()
        pltpu.make_async_copy(
            tbuf, out_hbm.at[pl.ds(base + B_PER_W - 2 * TAIL, TAIL), 0], otsem
        ).wait()

        def drain(k, carry):
            pltpu.make_async_copy(
                arena.at[pl.ds(0, MID)],
                out_hbm.at[pl.ds(base + EDGE, MID), 0],
                osems.at[k],
            ).wait()
            return carry

        lax.fori_loop(0, N_MID, drain, 0)

    return emb_kernel


_emb = _make_emb_kernel()


@jax.jit
def kernel(x, W, pe):
    return _emb(x.reshape(-1).astype(jnp.int32), W, pe)


# speedup vs baseline: 1.1105x; 1.1105x over previous
import functools
import math

import jax
import jax.numpy as jnp
from jax import lax
from jax.experimental import pallas as pl
from jax.experimental.pallas import tpu as pltpu
from jax.experimental.pallas import tpu_sc as plsc

D_MODEL = 1024
SEQ = 4096
LANES = 16
NUM_CORES = 2
NUM_SUBCORES = 16
NW = NUM_CORES * NUM_SUBCORES
B_PER_W = SEQ // NW
EDGE = 16
MID = 32
N_MID = 3
COLS = D_MODEL // LANES
CBLK = 16
SCALE = math.sqrt(float(D_MODEL))


def _make_emb_kernel():
    mesh = plsc.VectorSubcoreMesh(core_axis_name="c", subcore_axis_name="s")

    @functools.partial(
        pl.kernel,
        mesh=mesh,
        out_type=jax.ShapeDtypeStruct((SEQ, 1, D_MODEL), jnp.float32),
        scratch_types=[
            pltpu.VMEM((B_PER_W,), jnp.int32),
            pltpu.VMEM((1, D_MODEL), jnp.float32),
            pltpu.VMEM((EDGE, D_MODEL), jnp.float32),
            pltpu.VMEM((N_MID * MID, D_MODEL), jnp.float32),
            pltpu.SemaphoreType.DMA,
            pltpu.SemaphoreType.DMA,
            pltpu.SemaphoreType.DMA((N_MID,)),
            pltpu.SemaphoreType.DMA((N_MID,)),
        ],
    )
    def emb_kernel(x_hbm, w_hbm, pe_hbm, out_hbm, idx_v, pe_v, edge, arena,
                   esem, oesem, gsems, osems):
        wid = lax.axis_index("s") * NUM_CORES + lax.axis_index("c")
        base = wid * B_PER_W
        pltpu.sync_copy(x_hbm.at[pl.ds(base, B_PER_W)], idx_v)

        def compute(buf, row0, nrows):
            def blk(ib, carry):
                col0 = ib * (CBLK * LANES)
                pv = [
                    pe_v[0, pl.ds(col0 + i * LANES, LANES)]
                    for i in range(CBLK)
                ]

                def body(j, carry2):
                    for i in range(CBLK):
                        sl = pl.ds(col0 + i * LANES, LANES)
                        buf[row0 + j, sl] = buf[row0 + j, sl] * SCALE + pv[i]
                    return carry2

                lax.fori_loop(0, nrows, body, 0)
                return carry

            lax.fori_loop(0, COLS // CBLK, blk, 0)

        g_head = pltpu.async_copy(
            w_hbm.at[idx_v.at[pl.ds(0, EDGE)]], edge, esem
        )
        for k in range(N_MID):
            pltpu.async_copy(
                w_hbm.at[idx_v.at[pl.ds(EDGE + k * MID, MID)]],
                arena.at[pl.ds(k * MID, MID)],
                gsems.at[k],
            )
        pltpu.sync_copy(pe_hbm.at[pl.ds(0, 1), 0], pe_v)

        g_head.wait()
        compute(edge, 0, EDGE)
        pltpu.async_copy(edge, out_hbm.at[pl.ds(base, EDGE), 0], oesem)

        def mid(k, carry):
            row0 = k * MID
            pltpu.make_async_copy(
                w_hbm.at[idx_v.at[pl.ds(0, MID)]],
                arena.at[pl.ds(row0, MID)],
                gsems.at[k],
            ).wait()
            compute(arena, row0, MID)
            pltpu.async_copy(
                arena.at[pl.ds(row0, MID)],
                out_hbm.at[pl.ds(base + EDGE + row0, MID), 0],
                osems.at[k],
            )

            @pl.when(k == 0)
            def _():
                pltpu.make_async_copy(
                    edge, out_hbm.at[pl.ds(base, EDGE), 0], oesem
                ).wait()
                pltpu.async_copy(
                    w_hbm.at[idx_v.at[pl.ds(B_PER_W - EDGE, EDGE)]],
                    edge,
                    esem,
                )

            return carry

        lax.fori_loop(0, N_MID, mid, 0)

        pltpu.make_async_copy(
            w_hbm.at[idx_v.at[pl.ds(B_PER_W - EDGE, EDGE)]], edge, esem
        ).wait()
        compute(edge, 0, EDGE)
        pltpu.async_copy(
            edge, out_hbm.at[pl.ds(base + B_PER_W - EDGE, EDGE), 0], oesem
        ).wait()

        def drain(k, carry):
            pltpu.make_async_copy(
                arena.at[pl.ds(0, MID)],
                out_hbm.at[pl.ds(base + EDGE, MID), 0],
                osems.at[k],
            ).wait()
            return carry

        lax.fori_loop(0, N_MID, drain, 0)

    return emb_kernel


_emb = _make_emb_kernel()


@jax.jit
def kernel(x, W, pe):
    return _emb(x.reshape(-1).astype(jnp.int32), W, pe)


# speedup vs baseline: 1.1925x; 1.0738x over previous
import functools
import math

import jax
import jax.numpy as jnp
from jax import lax
from jax.experimental import pallas as pl
from jax.experimental.pallas import tpu as pltpu
from jax.experimental.pallas import tpu_sc as plsc

D_MODEL = 1024
SEQ = 4096
LANES = 16
NUM_CORES = 2
NUM_SUBCORES = 16
NW = NUM_CORES * NUM_SUBCORES
B_PER_W = SEQ // NW
EDGE = 16
MID = 32
N_MID = 3
COLS = D_MODEL // LANES
CBLK = 8
SCALE = math.sqrt(float(D_MODEL))


def _make_emb_kernel():
    mesh = plsc.VectorSubcoreMesh(core_axis_name="c", subcore_axis_name="s")

    @functools.partial(
        pl.kernel,
        mesh=mesh,
        out_type=jax.ShapeDtypeStruct((SEQ, 1, D_MODEL), jnp.float32),
        scratch_types=[
            pltpu.VMEM((B_PER_W,), jnp.int32),
            pltpu.VMEM((1, D_MODEL), jnp.float32),
            pltpu.VMEM((EDGE, D_MODEL), jnp.float32),
            pltpu.VMEM((N_MID * MID, D_MODEL), jnp.float32),
            pltpu.SemaphoreType.DMA,
            pltpu.SemaphoreType.DMA,
            pltpu.SemaphoreType.DMA((N_MID,)),
            pltpu.SemaphoreType.DMA((N_MID,)),
        ],
    )
    def emb_kernel(x_hbm, w_hbm, pe_hbm, out_hbm, idx_v, pe_v, edge, arena,
                   esem, oesem, gsems, osems):
        wid = lax.axis_index("s") * NUM_CORES + lax.axis_index("c")
        base = wid * B_PER_W
        pltpu.sync_copy(x_hbm.at[pl.ds(base, B_PER_W)], idx_v)

        def compute(buf, row0, nrows):
            def blk(ib, carry):
                col0 = ib * (CBLK * LANES)
                pv = [
                    pe_v[0, pl.ds(col0 + i * LANES, LANES)]
                    for i in range(CBLK)
                ]

                def body(j, carry2):
                    for i in range(CBLK):
                        sl = pl.ds(col0 + i * LANES, LANES)
                        buf[row0 + j, sl] = buf[row0 + j, sl] * SCALE + pv[i]
                    return carry2

                lax.fori_loop(0, nrows, body, 0)
                return carry

            lax.fori_loop(0, COLS // CBLK, blk, 0)

        g_head = pltpu.async_copy(
            w_hbm.at[idx_v.at[pl.ds(0, EDGE)]], edge, esem
        )
        for k in range(N_MID):
            pltpu.async_copy(
                w_hbm.at[idx_v.at[pl.ds(EDGE + k * MID, MID)]],
                arena.at[pl.ds(k * MID, MID)],
                gsems.at[k],
            )
        pltpu.sync_copy(pe_hbm.at[pl.ds(0, 1), 0], pe_v)

        g_head.wait()
        compute(edge, 0, EDGE)
        pltpu.async_copy(edge, out_hbm.at[pl.ds(base, EDGE), 0], oesem)

        def mid(k, carry):
            row0 = k * MID
            pltpu.make_async_copy(
                w_hbm.at[idx_v.at[pl.ds(0, MID)]],
                arena.at[pl.ds(row0, MID)],
                gsems.at[k],
            ).wait()
            compute(arena, row0, MID)
            pltpu.async_copy(
                arena.at[pl.ds(row0, MID)],
                out_hbm.at[pl.ds(base + EDGE + row0, MID), 0],
                osems.at[k],
            )

            @pl.when(k == 0)
            def _():
                pltpu.make_async_copy(
                    edge, out_hbm.at[pl.ds(base, EDGE), 0], oesem
                ).wait()
                pltpu.async_copy(
                    w_hbm.at[idx_v.at[pl.ds(B_PER_W - EDGE, EDGE)]],
                    edge,
                    esem,
                )

            return carry

        lax.fori_loop(0, N_MID, mid, 0)

        pltpu.make_async_copy(
            w_hbm.at[idx_v.at[pl.ds(B_PER_W - EDGE, EDGE)]], edge, esem
        ).wait()
        compute(edge, 0, EDGE)
        pltpu.async_copy(
            edge, out_hbm.at[pl.ds(base + B_PER_W - EDGE, EDGE), 0], oesem
        ).wait()

        def drain(k, carry):
            pltpu.make_async_copy(
                arena.at[pl.ds(0, MID)],
                out_hbm.at[pl.ds(base + EDGE, MID), 0],
                osems.at[k],
            ).wait()
            return carry

        lax.fori_loop(0, N_MID, drain, 0)

    return emb_kernel


_emb = _make_emb_kernel()


@jax.jit
def kernel(x, W, pe):
    return _emb(x.reshape(-1).astype(jnp.int32), W, pe)
